# baseline (speedup 1.0000x reference)
import functools
import math

import jax
import jax.numpy as jnp
from jax import lax
from jax.experimental import pallas as pl
from jax.experimental.pallas import tpu as pltpu
from jax.experimental.pallas import tpu_sc as plsc


def _pe_add_kernel(x_ref, pe_ref, o_ref, *, scale):
    o_ref[...] = x_ref[...] * scale + pe_ref[...]


def _kernel_tc(x, position_embeddings):
    B, S, D = x.shape
    scale = math.sqrt(D)
    BLK = 2048
    grid = (S // BLK, B)
    return pl.pallas_call(
        functools.partial(_pe_add_kernel, scale=scale),
        grid=grid,
        in_specs=[
            pl.BlockSpec((1, BLK, D), lambda s, b: (b, s, 0)),
            pl.BlockSpec((BLK, D), lambda s, b: (s, 0)),
        ],
        out_specs=pl.BlockSpec((1, BLK, D), lambda s, b: (b, s, 0)),
        out_shape=jax.ShapeDtypeStruct((B, S, D), x.dtype),
    )(x, position_embeddings[:S])


def _kernel_sc(x, position_embeddings):
    B, S, D = x.shape
    scale = math.sqrt(D)
    info = plsc.get_sparse_core_info()
    NC, NS, L = info.num_cores, info.num_subcores, info.num_lanes
    NW = NC * NS
    PR = S // NW
    TR = 8
    NG = PR // TR
    NBUF = 3

    mesh = plsc.VectorSubcoreMesh(core_axis_name="c", subcore_axis_name="s")

    @functools.partial(
        pl.kernel,
        mesh=mesh,
        out_type=jax.ShapeDtypeStruct((B, S, D), jnp.float32),
        scratch_types=(
            [pltpu.VMEM((B, TR, D), jnp.float32) for _ in range(NBUF)]
            + [pltpu.VMEM((TR, D), jnp.float32) for _ in range(NBUF)]
            + [pltpu.SemaphoreType.DMA for _ in range(2 * NBUF)]
        ),
    )
    def k(x_hbm, pe_hbm, out_hbm, *refs):
        xbufs = refs[0:NBUF]
        pebufs = refs[NBUF : 2 * NBUF]
        sin = refs[2 * NBUF : 3 * NBUF]
        sout = refs[3 * NBUF : 4 * NBUF]
        wid = lax.axis_index("s") * NC + lax.axis_index("c")
        pe_row0 = wid * PR

        def start_in(g):
            s = g % NBUF
            r0 = pe_row0 + g * TR
            return [
                pltpu.async_copy(pe_hbm.at[pl.ds(r0, TR)], pebufs[s], sin[s]),
                pltpu.async_copy(x_hbm.at[:, pl.ds(r0, TR)], xbufs[s], sin[s]),
            ]

        def start_out(g):
            s = g % NBUF
            r0 = pe_row0 + g * TR
            return [
                pltpu.async_copy(xbufs[s], out_hbm.at[:, pl.ds(r0, TR)], sout[s])
            ]

        def compute(g):
            s = g % NBUF
            xbuf = xbufs[s]
            pebuf = pebufs[s]

            @plsc.parallel_loop(0, D, step=L)
            def body(c):
                for r in range(TR):
                    pev = pebuf[r, pl.ds(c, L)]
                    for b in range(B):
                        xbuf[b, r, pl.ds(c, L)] = xbuf[b, r, pl.ds(c, L)] * scale + pev

        cps_in = {}
        cps_out = {}
        for g in range(min(2, NG)):
            cps_in[g] = start_in(g)
        for g in range(NG):
            for cp in cps_in[g]:
                cp.wait()
            if g == 0 and NG > 2:
                cps_in[2] = start_in(2)
            compute(g)
            cps_out[g] = start_out(g)
            if g >= 1 and g + 2 < NG:
                for cp in cps_out[g - 1]:
                    cp.wait()
                cps_in[g + 2] = start_in(g + 2)
        for g in range(max(0, NG - 3), NG):
            for cp in cps_out[g]:
                cp.wait()

    return k(x, position_embeddings[:S])


def kernel(x, position_embeddings):
    return _kernel_sc(x, position_embeddings)


# speedup vs baseline: 1.0141x; 1.0141x over previous
import functools
import math

import jax
import jax.numpy as jnp
from jax import lax
from jax.experimental import pallas as pl
from jax.experimental.pallas import tpu as pltpu
from jax.experimental.pallas import tpu_sc as plsc


def kernel(x, position_embeddings):
    B, S, D = x.shape
    scale = math.sqrt(D)
    info = plsc.get_sparse_core_info()
    NC, NS, L = info.num_cores, info.num_subcores, info.num_lanes
    NW = NC * NS
    PR = S // NW
    TR = 8
    NG = PR // TR
    NBUF = 3

    mesh = plsc.VectorSubcoreMesh(core_axis_name="c", subcore_axis_name="s")

    @functools.partial(
        pl.kernel,
        mesh=mesh,
        out_type=jax.ShapeDtypeStruct((B, S, D), jnp.float32),
        scratch_types=(
            [pltpu.VMEM((B, TR, D), jnp.float32) for _ in range(NBUF)]
            + [pltpu.VMEM((TR, D), jnp.float32) for _ in range(NBUF)]
            + [pltpu.SemaphoreType.DMA for _ in range(2 * NBUF)]
        ),
    )
    def k(x_hbm, pe_hbm, out_hbm, *refs):
        xbufs = refs[0:NBUF]
        pebufs = refs[NBUF : 2 * NBUF]
        sin = refs[2 * NBUF : 3 * NBUF]
        sout = refs[3 * NBUF : 4 * NBUF]
        wid = lax.axis_index("s") * NC + lax.axis_index("c")
        pe_row0 = wid * PR

        def start_in(g):
            s = g % NBUF
            r0 = pe_row0 + g * TR
            return [
                pltpu.async_copy(pe_hbm.at[pl.ds(r0, TR)], pebufs[s], sin[s]),
                pltpu.async_copy(x_hbm.at[:, pl.ds(r0, TR)], xbufs[s], sin[s]),
            ]

        def start_out(g):
            s = g % NBUF
            r0 = pe_row0 + g * TR
            return [
                pltpu.async_copy(xbufs[s], out_hbm.at[:, pl.ds(r0, TR)], sout[s])
            ]

        def compute(g):
            s = g % NBUF
            xbuf = xbufs[s]
            pebuf = pebufs[s]

            @plsc.parallel_loop(0, D, step=L)
            def body(c):
                for r in range(TR):
                    pev = pebuf[r, pl.ds(c, L)]
                    for b in range(B):
                        xbuf[b, r, pl.ds(c, L)] = xbuf[b, r, pl.ds(c, L)] * scale + pev

        cps_in = {}
        cps_out = {}
        for g in range(min(2, NG)):
            cps_in[g] = start_in(g)
        for g in range(NG):
            for cp in cps_in[g]:
                cp.wait()
            if g == 0 and NG > 2:
                cps_in[2] = start_in(2)
            if g >= 1 and g + 2 < NG:
                for cp in cps_out[g - 1]:
                    cp.wait()
                cps_in[g + 2] = start_in(g + 2)
            compute(g)
            cps_out[g] = start_out(g)
        for g in range(max(0, NG - 3), NG):
            for cp in cps_out[g]:
                cp.wait()

    return k(x, position_embeddings[:S])


# speedup vs baseline: 1.0151x; 1.0010x over previous
import functools
import math

import jax
import jax.numpy as jnp
from jax import lax
from jax.experimental import pallas as pl
from jax.experimental.pallas import tpu as pltpu
from jax.experimental.pallas import tpu_sc as plsc


def kernel(x, position_embeddings):
    B, S, D = x.shape
    scale = math.sqrt(D)
    info = plsc.get_sparse_core_info()
    NC, NS, L = info.num_cores, info.num_subcores, info.num_lanes
    NW = NC * NS
    PR = S // NW
    TR = 8
    NG = PR // TR
    NBUF = 3

    mesh = plsc.VectorSubcoreMesh(core_axis_name="c", subcore_axis_name="s")

    @functools.partial(
        pl.kernel,
        mesh=mesh,
        out_type=jax.ShapeDtypeStruct((B, S, D), jnp.float32),
        scratch_types=(
            [pltpu.VMEM((B, TR, D), jnp.float32) for _ in range(NBUF)]
            + [pltpu.VMEM((TR, D), jnp.float32) for _ in range(NBUF)]
            + [pltpu.SemaphoreType.DMA for _ in range(2 * NBUF)]
        ),
    )
    def k(x_hbm, pe_hbm, out_hbm, *refs):
        xbufs = refs[0:NBUF]
        pebufs = refs[NBUF : 2 * NBUF]
        sin = refs[2 * NBUF : 3 * NBUF]
        sout = refs[3 * NBUF : 4 * NBUF]
        wid = lax.axis_index("c") * NS + lax.axis_index("s")
        pe_row0 = wid * PR

        def start_in(g):
            s = g % NBUF
            r0 = pe_row0 + g * TR
            return [
                pltpu.async_copy(pe_hbm.at[pl.ds(r0, TR)], pebufs[s], sin[s]),
                pltpu.async_copy(x_hbm.at[:, pl.ds(r0, TR)], xbufs[s], sin[s]),
            ]

        def start_out(g):
            s = g % NBUF
            r0 = pe_row0 + g * TR
            return [
                pltpu.async_copy(xbufs[s], out_hbm.at[:, pl.ds(r0, TR)], sout[s])
            ]

        def compute(g):
            s = g % NBUF
            xbuf = xbufs[s]
            pebuf = pebufs[s]

            @plsc.parallel_loop(0, D, step=L)
            def body(c):
                for r in range(TR):
                    pev = pebuf[r, pl.ds(c, L)]
                    for b in range(B):
                        xbuf[b, r, pl.ds(c, L)] = xbuf[b, r, pl.ds(c, L)] * scale + pev

        cps_in = {}
        cps_out = {}
        for g in range(min(2, NG)):
            cps_in[g] = start_in(g)
        for g in range(NG):
            for cp in cps_in[g]:
                cp.wait()
            if g == 0 and NG > 2:
                cps_in[2] = start_in(2)
            if g >= 1 and g + 2 < NG:
                for cp in cps_out[g - 1]:
                    cp.wait()
                cps_in[g + 2] = start_in(g + 2)
            compute(g)
            cps_out[g] = start_out(g)
        for g in range(max(0, NG - 3), NG):
            for cp in cps_out[g]:
                cp.wait()

    return k(x, position_embeddings[:S])
